# baseline (speedup 1.0000x reference)
import functools

import jax
import jax.numpy as jnp
from jax import lax
from jax.experimental import pallas as pl
from jax.experimental.pallas import tpu as pltpu
from jax.experimental.pallas import tpu_sc as plsc

_VOCAB = 1000000
_EMBED = 64
_MODES = 32
_QUART = _VOCAB // 4


_BLK = 2000


def _transform_body(fr0, fi0, fr1, fi1, fr2, fi2, fr3, fi3, m_ref, b_ref, out_ref):
    def words(fr_ref, fi_ref):
        x = jnp.concatenate(
            [fr_ref[...].astype(jnp.bfloat16), fi_ref[...].astype(jnp.bfloat16)],
            axis=1,
        )
        acc = jnp.dot(x, m_ref[...], preferred_element_type=jnp.float32)
        acc = acc + b_ref[...]
        lo = lax.bitcast_convert_type(acc[:, :_MODES].astype(jnp.bfloat16), jnp.uint16)
        hi = lax.bitcast_convert_type(acc[:, _MODES:].astype(jnp.bfloat16), jnp.uint16)
        return (lo.astype(jnp.uint32) | (hi.astype(jnp.uint32) << 16)).astype(jnp.int32)

    out_ref[...] = jnp.concatenate(
        [words(fr0, fi0), words(fr1, fi1), words(fr2, fi2), words(fr3, fi3)], axis=1
    )


def _build_table(freq_real, freq_imag, a_mat, bias):
    nb = _QUART // _BLK

    def fspec(q):
        return pl.BlockSpec((_BLK, _MODES), lambda i, q=q: (q * nb + i, 0))

    return pl.pallas_call(
        _transform_body,
        grid=(nb,),
        in_specs=[
            fspec(0), fspec(0), fspec(1), fspec(1),
            fspec(2), fspec(2), fspec(3), fspec(3),
            pl.BlockSpec((2 * _MODES, _EMBED), lambda i: (0, 0)),
            pl.BlockSpec((1, _EMBED), lambda i: (0, 0)),
        ],
        out_specs=pl.BlockSpec((_BLK, 128), lambda i: (i, 0)),
        out_shape=jax.ShapeDtypeStruct((_QUART, 128), jnp.int32),
    )(freq_real, freq_imag, freq_real, freq_imag,
      freq_real, freq_imag, freq_real, freq_imag, a_mat, bias)


_NC, _NS = 2, 16
_NW = _NC * _NS
_CH = 128
_L = 16


def _make_gather(n_tok):
    per_w = n_tok // _NW
    n_ch = per_w // _CH
    mesh = plsc.VectorSubcoreMesh(core_axis_name="c", subcore_axis_name="s")
    mask = jnp.int32(-65536)

    @functools.partial(
        pl.kernel,
        mesh=mesh,
        compiler_params=pltpu.CompilerParams(
            use_tc_tiling_on_sc=False, needs_layout_passes=False
        ),
        out_type=jax.ShapeDtypeStruct((n_tok, _EMBED), jnp.float32),
        scratch_types=[
            pltpu.VMEM((_CH,), jnp.int32),
            pltpu.VMEM((_CH + _L,), jnp.int32),
            pltpu.VMEM((_CH, 128), jnp.int32),
            pltpu.VMEM((_CH, _EMBED), jnp.float32),
            pltpu.SemaphoreType.DMA,
        ],
    )
    def gather_k(table_hbm, row_hbm, qb_hbm, out_hbm, row_v, qb_v, rows_v, dec_v, sem):
        wid = lax.axis_index("s") * _NC + lax.axis_index("c")
        base = wid * per_w

        def chunk(i, carry):
            off = base + i * _CH
            pltpu.sync_copy(row_hbm.at[pl.ds(off, _CH)], row_v)
            pltpu.sync_copy(qb_hbm.at[pl.ds(off, _CH)], qb_v.at[pl.ds(0, _CH)])
            pltpu.async_copy(table_hbm.at[row_v], rows_v, sem).wait()

            def row(j, c2):
                qb = qb_v[pl.ds(j, _L)][0]
                for k in range(_MODES // _L):
                    word = rows_v[j, pl.ds(qb + k * _L, _L)]
                    lo = plsc.bitcast(word << 16, jnp.float32)
                    hi = plsc.bitcast(word & mask, jnp.float32)
                    dec_v[j, pl.ds(2 * k * _L, _L)] = lo
                    dec_v[j, pl.ds((2 * k + 1) * _L, _L)] = hi
                return c2

            lax.fori_loop(0, _CH, row, 0)
            pltpu.sync_copy(dec_v, out_hbm.at[pl.ds(off, _CH)])
            return carry

        lax.fori_loop(0, n_ch, chunk, 0)

    return gather_k


def kernel(tokens, freq_real, freq_imag, mode_weights, phase, W, b):
    w = jax.nn.softplus(mode_weights)
    c = jnp.cos(phase)
    s = jnp.sin(phase)
    w1t = W[:, :_MODES].T
    w2t = W[:, _MODES:].T
    a_real = (w * c)[:, None] * w1t + (w * s)[:, None] * w2t
    a_imag = (w * c)[:, None] * w2t - (w * s)[:, None] * w1t
    a_mat = jnp.concatenate([a_real, a_imag], axis=0)
    bias = b.reshape(1, _EMBED)
    perm = jnp.concatenate(
        [jnp.arange(0, 16), jnp.arange(32, 48), jnp.arange(16, 32), jnp.arange(48, 64)]
    )
    a_mat = a_mat[:, perm].astype(jnp.bfloat16)
    bias = bias[:, perm]

    table = _build_table(freq_real, freq_imag, a_mat, bias)

    bsz, tsz = tokens.shape
    idx = tokens.reshape(-1).astype(jnp.int32)
    rows = idx % _QUART
    qbase = (idx // _QUART) * _MODES
    out = _make_gather(bsz * tsz)(table, rows, qbase)
    return out.reshape(bsz, tsz, _EMBED)


# speedup vs baseline: 1.0717x; 1.0717x over previous
import functools

import jax
import jax.numpy as jnp
from jax import lax
from jax.experimental import pallas as pl
from jax.experimental.pallas import tpu as pltpu
from jax.experimental.pallas import tpu_sc as plsc

_VOCAB = 1000000
_EMBED = 64
_MODES = 32


_BLK = 8000


def _transform_body(fr_ref, fi_ref, m_ref, b_ref, out_ref):
    x = jnp.concatenate(
        [fr_ref[...].astype(jnp.bfloat16), fi_ref[...].astype(jnp.bfloat16)],
        axis=1,
    )
    acc = jnp.dot(x, m_ref[...], preferred_element_type=jnp.float32)
    acc = acc + b_ref[...]
    lo = lax.bitcast_convert_type(acc[:, :_MODES].astype(jnp.bfloat16), jnp.uint16)
    hi = lax.bitcast_convert_type(acc[:, _MODES:].astype(jnp.bfloat16), jnp.uint16)
    word = lo.astype(jnp.uint32) | (hi.astype(jnp.uint32) << 16)
    out_ref[...] = word.astype(jnp.int32)


def _build_table(freq_real, freq_imag, a_mat, bias):
    grid = (_VOCAB // _BLK,)
    return pl.pallas_call(
        _transform_body,
        grid=grid,
        in_specs=[
            pl.BlockSpec((_BLK, _MODES), lambda i: (i, 0)),
            pl.BlockSpec((_BLK, _MODES), lambda i: (i, 0)),
            pl.BlockSpec((2 * _MODES, _EMBED), lambda i: (0, 0)),
            pl.BlockSpec((1, _EMBED), lambda i: (0, 0)),
        ],
        out_specs=pl.BlockSpec((_BLK, _MODES), lambda i: (i, 0)),
        out_shape=jax.ShapeDtypeStruct((_VOCAB, _MODES), jnp.int32),
    )(freq_real, freq_imag, a_mat, bias)


_NC, _NS = 2, 16
_NW = _NC * _NS
_CH = 128
_L = 16


def _make_gather(n_tok):
    per_w = n_tok // _NW
    n_ch = per_w // _CH
    mesh = plsc.VectorSubcoreMesh(core_axis_name="c", subcore_axis_name="s")
    mask = jnp.uint32(0xFFFF0000).astype(jnp.int32)

    @functools.partial(
        pl.kernel,
        mesh=mesh,
        compiler_params=pltpu.CompilerParams(
            use_tc_tiling_on_sc=False, needs_layout_passes=False
        ),
        out_type=jax.ShapeDtypeStruct((n_tok, _EMBED), jnp.float32),
        scratch_types=[
            pltpu.VMEM((_CH,), jnp.int32),
            pltpu.VMEM((_CH, _MODES), jnp.int32),
            pltpu.VMEM((_CH, _EMBED), jnp.float32),
            pltpu.SemaphoreType.DMA,
        ],
    )
    def gather_k(table_hbm, idx_hbm, out_hbm, idx_v, rows_v, dec_v, sem):
        wid = lax.axis_index("s") * _NC + lax.axis_index("c")
        base = wid * per_w

        def chunk(i, carry):
            off = base + i * _CH
            pltpu.sync_copy(idx_hbm.at[pl.ds(off, _CH)], idx_v)
            pltpu.async_copy(table_hbm.at[idx_v], rows_v, sem).wait()

            def row(j, c2):
                for k in range(_MODES // _L):
                    word = rows_v[j, pl.ds(k * _L, _L)]
                    lo = plsc.bitcast(word << 16, jnp.float32)
                    hi = plsc.bitcast(word & mask, jnp.float32)
                    dec_v[j, pl.ds(2 * k * _L, _L)] = lo
                    dec_v[j, pl.ds((2 * k + 1) * _L, _L)] = hi
                return c2

            lax.fori_loop(0, _CH, row, 0)
            pltpu.sync_copy(dec_v, out_hbm.at[pl.ds(off, _CH)])
            return carry

        lax.fori_loop(0, n_ch, chunk, 0)

    return gather_k


def kernel(tokens, freq_real, freq_imag, mode_weights, phase, W, b):
    w = jax.nn.softplus(mode_weights)
    c = jnp.cos(phase)
    s = jnp.sin(phase)
    w1t = W[:, :_MODES].T
    w2t = W[:, _MODES:].T
    a_real = (w * c)[:, None] * w1t + (w * s)[:, None] * w2t
    a_imag = (w * c)[:, None] * w2t - (w * s)[:, None] * w1t
    a_mat = jnp.concatenate([a_real, a_imag], axis=0)
    bias = b.reshape(1, _EMBED)
    perm = jnp.concatenate(
        [jnp.arange(0, 16), jnp.arange(32, 48), jnp.arange(16, 32), jnp.arange(48, 64)]
    )
    a_mat = a_mat[:, perm].astype(jnp.bfloat16)
    bias = bias[:, perm]

    table = _build_table(freq_real, freq_imag, a_mat, bias)

    bsz, tsz = tokens.shape
    idx = tokens.reshape(-1).astype(jnp.int32)
    out = _make_gather(bsz * tsz)(table, idx)
    return out.reshape(bsz, tsz, _EMBED)


# speedup vs baseline: 1.1428x; 1.0664x over previous
import functools

import jax
import jax.numpy as jnp
from jax import lax
from jax.experimental import pallas as pl
from jax.experimental.pallas import tpu as pltpu
from jax.experimental.pallas import tpu_sc as plsc

_VOCAB = 1000000
_EMBED = 64
_MODES = 32


_BLK = 8000


def _transform_body(fr_ref, fi_ref, m_ref, b_ref, out_ref):
    x = jnp.concatenate(
        [fr_ref[...].astype(jnp.bfloat16), fi_ref[...].astype(jnp.bfloat16)],
        axis=1,
    )
    acc = jnp.dot(x, m_ref[...], preferred_element_type=jnp.float32)
    out_ref[...] = acc + b_ref[...]


def _build_table(freq_real, freq_imag, a_mat, bias):
    grid = (_VOCAB // _BLK,)
    return pl.pallas_call(
        _transform_body,
        grid=grid,
        in_specs=[
            pl.BlockSpec((_BLK, _MODES), lambda i: (i, 0)),
            pl.BlockSpec((_BLK, _MODES), lambda i: (i, 0)),
            pl.BlockSpec((2 * _MODES, _EMBED), lambda i: (0, 0)),
            pl.BlockSpec((1, _EMBED), lambda i: (0, 0)),
        ],
        out_specs=pl.BlockSpec((_BLK, _EMBED), lambda i: (i, 0)),
        out_shape=jax.ShapeDtypeStruct((_VOCAB, _EMBED), jnp.float32),
    )(freq_real, freq_imag, a_mat, bias)


_NC, _NS = 2, 16
_NW = _NC * _NS
_CH = 128


def _make_gather(n_tok):
    per_w = n_tok // _NW
    n_ch = per_w // _CH
    mesh = plsc.VectorSubcoreMesh(core_axis_name="c", subcore_axis_name="s")

    @functools.partial(
        pl.kernel,
        mesh=mesh,
        compiler_params=pltpu.CompilerParams(
            use_tc_tiling_on_sc=False, needs_layout_passes=False
        ),
        out_type=jax.ShapeDtypeStruct((n_tok, _EMBED), jnp.float32),
        scratch_types=[
            pltpu.VMEM((_CH,), jnp.int32),
            pltpu.VMEM((_CH, _EMBED), jnp.float32),
            pltpu.SemaphoreType.DMA,
        ],
    )
    def gather_k(table_hbm, idx_hbm, out_hbm, idx_v, rows_v, sem):
        wid = lax.axis_index("s") * _NC + lax.axis_index("c")
        base = wid * per_w

        def chunk(i, carry):
            off = base + i * _CH
            pltpu.sync_copy(idx_hbm.at[pl.ds(off, _CH)], idx_v)
            pltpu.async_copy(table_hbm.at[idx_v], rows_v, sem).wait()
            pltpu.sync_copy(rows_v, out_hbm.at[pl.ds(off, _CH)])
            return carry

        lax.fori_loop(0, n_ch, chunk, 0)

    return gather_k


def kernel(tokens, freq_real, freq_imag, mode_weights, phase, W, b):
    w = jax.nn.softplus(mode_weights)
    c = jnp.cos(phase)
    s = jnp.sin(phase)
    w1t = W[:, :_MODES].T
    w2t = W[:, _MODES:].T
    a_real = (w * c)[:, None] * w1t + (w * s)[:, None] * w2t
    a_imag = (w * c)[:, None] * w2t - (w * s)[:, None] * w1t
    a_mat = jnp.concatenate([a_real, a_imag], axis=0).astype(jnp.bfloat16)
    bias = b.reshape(1, _EMBED)

    table = _build_table(freq_real, freq_imag, a_mat, bias)

    bsz, tsz = tokens.shape
    idx = tokens.reshape(-1).astype(jnp.int32)
    out = _make_gather(bsz * tsz)(table, idx)
    return out.reshape(bsz, tsz, _EMBED)
